# baseline (device time: 34396 ns/iter reference)
import functools

import jax
import jax.numpy as jnp
from jax import lax
from jax.experimental import pallas as pl
from jax.experimental.pallas import tpu as pltpu

N_DEV = 4
B_LOC = 2
SQ = 128
SKV = 128
H_LOC = 8
DH = 64
D_LOC = H_LOC * DH
D_MODEL = 512


def kernel(x, Wq, Wo, K_ext, V_ext):
    my = lax.axis_index("i")
    K_my = jnp.transpose(
        lax.dynamic_slice_in_dim(K_ext, my * H_LOC, H_LOC, axis=2), (0, 2, 1, 3)
    )
    V_my = jnp.transpose(
        lax.dynamic_slice_in_dim(V_ext, my * H_LOC, H_LOC, axis=2), (0, 2, 1, 3)
    )

    def body(x_ref, wq_ref, wo_ref, k_ref, v_ref, out_ref,
             xbuf, accs, accr, oref, wqb, wob, kb, vb,
             xsend, xrecv, asend, arecv):
        me = lax.axis_index("i")
        left = lax.rem(me + N_DEV - 1, N_DEV)
        right = lax.rem(me + 1, N_DEV)

        bsem = pltpu.get_barrier_semaphore()
        pl.semaphore_signal(bsem, inc=1, device_id=(left,),
                            device_id_type=pl.DeviceIdType.MESH)
        pl.semaphore_signal(bsem, inc=1, device_id=(right,),
                            device_id_type=pl.DeviceIdType.MESH)
        pl.semaphore_wait(bsem, 2)

        wqb[...] = wq_ref[...].astype(jnp.bfloat16)
        wob[...] = wo_ref[...].astype(jnp.bfloat16)
        kb[...] = k_ref[...].astype(jnp.bfloat16)
        vb[...] = jnp.zeros_like(vb)
        vb[:, :, :, 0:DH] = v_ref[...].astype(jnp.bfloat16)
        vb[:, :, :, DH:DH + 1] = jnp.ones(
            (N_DEV * B_LOC, H_LOC, SKV, 1), jnp.bfloat16
        )
        xbuf[0] = x_ref[...].astype(jnp.bfloat16)

        def make_x(t):
            return pltpu.make_async_remote_copy(
                src_ref=xbuf.at[t], dst_ref=xbuf.at[t + 1],
                send_sem=xsend.at[t], recv_sem=xrecv.at[t],
                device_id=(right,), device_id_type=pl.DeviceIdType.MESH,
            )

        def make_a(t):
            return pltpu.make_async_remote_copy(
                src_ref=accs.at[t], dst_ref=accr.at[t],
                send_sem=asend.at[t], recv_sem=arecv.at[t],
                device_id=(right,), device_id_type=pl.DeviceIdType.MESH,
            )

        def compute_contrib(c, xslot):
            xm = xbuf[xslot].reshape(B_LOC * SQ, D_MODEL)
            q_all = (
                lax.dot(xm, wqb[...], preferred_element_type=jnp.float32)
                * 0.125
            ).astype(jnp.bfloat16)
            for bb in range(B_LOC):
                gb = c * B_LOC + bb
                for hh in range(H_LOC):
                    q = q_all[bb * SQ:(bb + 1) * SQ, hh * DH:(hh + 1) * DH]
                    k = kb[gb, hh]
                    s = lax.dot_general(
                        q, k, (((1,), (1,)), ((), ())),
                        preferred_element_type=jnp.float32,
                    )
                    p = jnp.exp(s).astype(jnp.bfloat16)
                    ov = lax.dot(
                        p, vb[gb, hh], preferred_element_type=jnp.float32
                    )
                    o = ov[:, 0:DH] * (1.0 / ov[:, DH:DH + 1])
                    oref[bb * SQ:(bb + 1) * SQ, hh * DH:(hh + 1) * DH] = (
                        o.astype(jnp.bfloat16)
                    )
            return lax.dot(
                oref[...], wob[...], preferred_element_type=jnp.float32
            )

        x_rdmas = [make_x(0)]
        x_rdmas[0].start()
        out_ref[...] = compute_contrib(me, 0).reshape(B_LOC, SQ, D_MODEL)

        a_rdmas = []
        for t in range(N_DEV - 1):
            x_rdmas[t].wait_recv()
            if t < N_DEV - 2:
                r = make_x(t + 1)
                r.start()
                x_rdmas.append(r)
            c = lax.rem(me + N_DEV - 1 - t, N_DEV)
            val = compute_contrib(c, t + 1)
            if t > 0:
                a_rdmas[t - 1].wait_recv()
                val = val + accr[t - 1].reshape(
                    B_LOC * SQ, D_MODEL
                ).astype(jnp.float32)
            accs[t] = val.astype(jnp.bfloat16).reshape(B_LOC, SQ, D_MODEL)
            ra = make_a(t)
            ra.start()
            a_rdmas.append(ra)

        a_rdmas[N_DEV - 2].wait_recv()
        out_ref[...] = out_ref[...] + accr[N_DEV - 2].astype(jnp.float32)

        for r in x_rdmas:
            r.wait_send()
        for r in a_rdmas:
            r.wait_send()

    return pl.pallas_call(
        body,
        out_shape=jax.ShapeDtypeStruct((B_LOC, SQ, D_MODEL), jnp.float32),
        in_specs=[pl.BlockSpec(memory_space=pltpu.VMEM)] * 5,
        out_specs=pl.BlockSpec(memory_space=pltpu.VMEM),
        scratch_shapes=[
            pltpu.VMEM((N_DEV, B_LOC, SQ, D_MODEL), jnp.bfloat16),
            pltpu.VMEM((N_DEV - 1, B_LOC, SQ, D_MODEL), jnp.bfloat16),
            pltpu.VMEM((N_DEV - 1, B_LOC, SQ, D_MODEL), jnp.bfloat16),
            pltpu.VMEM((B_LOC * SQ, D_MODEL), jnp.bfloat16),
            pltpu.VMEM((D_MODEL, D_LOC), jnp.bfloat16),
            pltpu.VMEM((D_LOC, D_MODEL), jnp.bfloat16),
            pltpu.VMEM((N_DEV * B_LOC, H_LOC, SKV, DH), jnp.bfloat16),
            pltpu.VMEM((N_DEV * B_LOC, H_LOC, SKV, 2 * DH), jnp.bfloat16),
            pltpu.SemaphoreType.DMA((N_DEV - 1,)),
            pltpu.SemaphoreType.DMA((N_DEV - 1,)),
            pltpu.SemaphoreType.DMA((N_DEV - 1,)),
            pltpu.SemaphoreType.DMA((N_DEV - 1,)),
        ],
        compiler_params=pltpu.CompilerParams(collective_id=0),
    )(x, Wq, Wo, K_my, V_my)


# device time: 22960 ns/iter; 1.4981x vs baseline; 1.4981x over previous
import os

import jax
import jax.numpy as jnp
from jax import lax
from jax.experimental import pallas as pl
from jax.experimental.pallas import tpu as pltpu

N_DEV = 4
B_LOC = 2
SQ = 128
SKV = 128
H_LOC = 8
DH = 64
D_MODEL = 512


def kernel(x, Wq, Wo, K_ext, V_ext):
    my = lax.axis_index("i")
    f16 = jnp.bfloat16
    K_my = jnp.transpose(
        lax.dynamic_slice_in_dim(K_ext, my * H_LOC, H_LOC, axis=2), (0, 2, 1, 3)
    ).astype(f16)
    V_my = jnp.transpose(
        lax.dynamic_slice_in_dim(V_ext, my * H_LOC, H_LOC, axis=2), (0, 2, 1, 3)
    ).astype(f16)
    Wq_h = jnp.transpose(
        Wq.reshape(D_MODEL, H_LOC, DH), (1, 0, 2)
    ).astype(f16)
    Wo_h = Wo.reshape(H_LOC, DH, D_MODEL).astype(f16)
    x_b = x.astype(f16)

    def body(x_ref, wq_ref, wo_ref, k_ref, v_ref, out_ref,
             xbuf, accs, accr, ohm,
             xsend, xrecv, asend, arecv):
        me = lax.axis_index("i")
        left = lax.rem(me + N_DEV - 1, N_DEV)
        right = lax.rem(me + 1, N_DEV)

        bsem = pltpu.get_barrier_semaphore()
        pl.semaphore_signal(bsem, inc=1, device_id=(left,),
                            device_id_type=pl.DeviceIdType.MESH)
        pl.semaphore_signal(bsem, inc=1, device_id=(right,),
                            device_id_type=pl.DeviceIdType.MESH)
        pl.semaphore_wait(bsem, 2)

        xbuf[0] = x_ref[...]

        def make_x(t):
            return pltpu.make_async_remote_copy(
                src_ref=xbuf.at[t], dst_ref=xbuf.at[t + 1],
                send_sem=xsend.at[t], recv_sem=xrecv.at[t],
                device_id=(right,), device_id_type=pl.DeviceIdType.MESH,
            )

        def make_a(t):
            return pltpu.make_async_remote_copy(
                src_ref=accs.at[t], dst_ref=accr.at[t],
                send_sem=asend.at[t], recv_sem=arecv.at[t],
                device_id=(right,), device_id_type=pl.DeviceIdType.MESH,
            )

        def compute_contrib(c, xslot):
            xm = xbuf[xslot].reshape(B_LOC * SQ, D_MODEL)
            for hh in range(H_LOC):
                qh = (
                    lax.dot(xm, wq_ref[hh], preferred_element_type=jnp.float32)
                    * 0.125
                ).astype(f16)
                for bb in range(B_LOC):
                    gb = c * B_LOC + bb
                    q = qh[bb * SQ:(bb + 1) * SQ]
                    s = lax.dot_general(
                        q, k_ref[gb, hh], (((1,), (1,)), ((), ())),
                        preferred_element_type=jnp.float32,
                    )
                    p = jnp.exp(s)
                    lsum = jnp.sum(p, axis=1, keepdims=True)
                    o = lax.dot(
                        p.astype(f16), v_ref[gb, hh],
                        preferred_element_type=jnp.float32,
                    ) * (1.0 / lsum)
                    ohm[hh, bb * SQ:(bb + 1) * SQ] = o.astype(f16)
            acc = lax.dot(
                ohm[0], wo_ref[0], preferred_element_type=jnp.float32
            )
            for hh in range(1, H_LOC):
                acc = acc + lax.dot(
                    ohm[hh], wo_ref[hh], preferred_element_type=jnp.float32
                )
            return acc

        if os.environ.get("ABLATE_RDMA"):
            acc = compute_contrib(me, 0)
            for k in range(1, N_DEV):
                acc = acc + compute_contrib(lax.rem(me + k, N_DEV), 0)
            out_ref[...] = acc.reshape(B_LOC, SQ, D_MODEL)
            return

        x_rdmas = [make_x(0)]
        x_rdmas[0].start()
        out_ref[...] = compute_contrib(me, 0).reshape(B_LOC, SQ, D_MODEL)

        a_rdmas = []
        for t in range(N_DEV - 1):
            x_rdmas[t].wait_recv()
            if t < N_DEV - 2:
                r = make_x(t + 1)
                r.start()
                x_rdmas.append(r)
            c = lax.rem(me + N_DEV - 1 - t, N_DEV)
            val = compute_contrib(c, t + 1)
            if t > 0:
                a_rdmas[t - 1].wait_recv()
                val = val + accr[t - 1].reshape(
                    B_LOC * SQ, D_MODEL
                ).astype(jnp.float32)
            accs[t] = val.astype(f16).reshape(B_LOC, SQ, D_MODEL)
            ra = make_a(t)
            ra.start()
            a_rdmas.append(ra)

        a_rdmas[N_DEV - 2].wait_recv()
        out_ref[...] = out_ref[...] + accr[N_DEV - 2].astype(jnp.float32)

        for r in x_rdmas:
            r.wait_send()
        for r in a_rdmas:
            r.wait_send()

    return pl.pallas_call(
        body,
        out_shape=jax.ShapeDtypeStruct((B_LOC, SQ, D_MODEL), jnp.float32),
        in_specs=[pl.BlockSpec(memory_space=pltpu.VMEM)] * 5,
        out_specs=pl.BlockSpec(memory_space=pltpu.VMEM),
        scratch_shapes=[
            pltpu.VMEM((N_DEV, B_LOC, SQ, D_MODEL), f16),
            pltpu.VMEM((N_DEV - 1, B_LOC, SQ, D_MODEL), f16),
            pltpu.VMEM((N_DEV - 1, B_LOC, SQ, D_MODEL), f16),
            pltpu.VMEM((H_LOC, B_LOC * SQ, DH), f16),
            pltpu.SemaphoreType.DMA((N_DEV - 1,)),
            pltpu.SemaphoreType.DMA((N_DEV - 1,)),
            pltpu.SemaphoreType.DMA((N_DEV - 1,)),
            pltpu.SemaphoreType.DMA((N_DEV - 1,)),
        ],
        compiler_params=pltpu.CompilerParams(collective_id=0),
    )(x_b, Wq_h, Wo_h, K_my, V_my)


# device time: 17143 ns/iter; 2.0064x vs baseline; 1.3393x over previous
import os

import jax
import jax.numpy as jnp
from jax import lax
from jax.experimental import pallas as pl
from jax.experimental.pallas import tpu as pltpu

N_DEV = 4
B_LOC = 2
SQ = 128
SKV = 128
H_LOC = 8
DH = 64
D_MODEL = 512


def kernel(x, Wq, Wo, K_ext, V_ext):
    my = lax.axis_index("i")
    f16 = jnp.bfloat16
    K_my = jnp.transpose(
        lax.dynamic_slice_in_dim(K_ext, my * H_LOC, H_LOC, axis=2), (0, 2, 1, 3)
    ).astype(f16)
    V_my = jnp.transpose(
        lax.dynamic_slice_in_dim(V_ext, my * H_LOC, H_LOC, axis=2), (0, 2, 1, 3)
    ).astype(f16)
    Wq_h = jnp.transpose(
        Wq.reshape(D_MODEL, H_LOC, DH), (1, 0, 2)
    ).astype(f16)
    Wo_h = Wo.reshape(H_LOC, DH, D_MODEL).astype(f16)
    x_b = x.astype(f16)

    def body(x_ref, wq_ref, wo_ref, k_ref, v_ref, out_ref,
             xbuf, accs, accr, ohm, qh,
             xsend, xrecv, asend, arecv):
        me = lax.axis_index("i")
        left = lax.rem(me + N_DEV - 1, N_DEV)
        right = lax.rem(me + 1, N_DEV)

        bsem = pltpu.get_barrier_semaphore()
        pl.semaphore_signal(bsem, inc=1, device_id=(left,),
                            device_id_type=pl.DeviceIdType.MESH)
        pl.semaphore_signal(bsem, inc=1, device_id=(right,),
                            device_id_type=pl.DeviceIdType.MESH)
        pl.semaphore_wait(bsem, 2)

        xbuf[0] = x_ref[...]

        def make_x(t):
            return pltpu.make_async_remote_copy(
                src_ref=xbuf.at[t], dst_ref=xbuf.at[t + 1],
                send_sem=xsend.at[t], recv_sem=xrecv.at[t],
                device_id=(right,), device_id_type=pl.DeviceIdType.MESH,
            )

        def make_a(t):
            return pltpu.make_async_remote_copy(
                src_ref=accs.at[t], dst_ref=accr.at[t],
                send_sem=asend.at[t], recv_sem=arecv.at[t],
                device_id=(right,), device_id_type=pl.DeviceIdType.MESH,
            )

        def compute_contrib(c, xslot):
            xm = xbuf[xslot].reshape(B_LOC * SQ, D_MODEL)
            for hh in range(H_LOC):
                qh[hh] = (
                    lax.dot(xm, wq_ref[hh], preferred_element_type=jnp.float32)
                    * 0.125
                ).astype(f16)
            for bb in range(B_LOC):
                gb = c * B_LOC + bb
                qb = qh[:, bb * SQ:(bb + 1) * SQ, :]
                s = lax.dot_general(
                    qb, k_ref[gb], (((2,), (2,)), ((0,), (0,))),
                    preferred_element_type=jnp.float32,
                )
                p = jnp.exp(s)
                lsum = jnp.sum(p, axis=2, keepdims=True)
                o = lax.dot_general(
                    p.astype(f16), v_ref[gb], (((2,), (1,)), ((0,), (0,))),
                    preferred_element_type=jnp.float32,
                ) * (1.0 / lsum)
                ohm[:, bb * SQ:(bb + 1) * SQ, :] = o.astype(f16)
            acc = lax.dot(
                ohm[0], wo_ref[0], preferred_element_type=jnp.float32
            )
            for hh in range(1, H_LOC):
                acc = acc + lax.dot(
                    ohm[hh], wo_ref[hh], preferred_element_type=jnp.float32
                )
            return acc

        if os.environ.get("ABLATE_RDMA"):
            acc = compute_contrib(me, 0)
            for k in range(1, N_DEV):
                acc = acc + compute_contrib(lax.rem(me + k, N_DEV), 0)
            out_ref[...] = acc.reshape(B_LOC, SQ, D_MODEL)
            return

        x_rdmas = [make_x(0)]
        x_rdmas[0].start()
        out_ref[...] = compute_contrib(me, 0).reshape(B_LOC, SQ, D_MODEL)

        a_rdmas = []
        for t in range(N_DEV - 1):
            x_rdmas[t].wait_recv()
            if t < N_DEV - 2:
                r = make_x(t + 1)
                r.start()
                x_rdmas.append(r)
            c = lax.rem(me + N_DEV - 1 - t, N_DEV)
            val = compute_contrib(c, t + 1)
            if t > 0:
                a_rdmas[t - 1].wait_recv()
                val = val + accr[t - 1].reshape(
                    B_LOC * SQ, D_MODEL
                ).astype(jnp.float32)
            accs[t] = val.astype(f16).reshape(B_LOC, SQ, D_MODEL)
            ra = make_a(t)
            ra.start()
            a_rdmas.append(ra)

        a_rdmas[N_DEV - 2].wait_recv()
        out_ref[...] = out_ref[...] + accr[N_DEV - 2].astype(jnp.float32)

        for r in x_rdmas:
            r.wait_send()
        for r in a_rdmas:
            r.wait_send()

    return pl.pallas_call(
        body,
        out_shape=jax.ShapeDtypeStruct((B_LOC, SQ, D_MODEL), jnp.float32),
        in_specs=[pl.BlockSpec(memory_space=pltpu.VMEM)] * 5,
        out_specs=pl.BlockSpec(memory_space=pltpu.VMEM),
        scratch_shapes=[
            pltpu.VMEM((N_DEV, B_LOC, SQ, D_MODEL), f16),
            pltpu.VMEM((N_DEV - 1, B_LOC, SQ, D_MODEL), f16),
            pltpu.VMEM((N_DEV - 1, B_LOC, SQ, D_MODEL), f16),
            pltpu.VMEM((H_LOC, B_LOC * SQ, DH), f16),
            pltpu.VMEM((H_LOC, B_LOC * SQ, DH), f16),
            pltpu.SemaphoreType.DMA((N_DEV - 1,)),
            pltpu.SemaphoreType.DMA((N_DEV - 1,)),
            pltpu.SemaphoreType.DMA((N_DEV - 1,)),
            pltpu.SemaphoreType.DMA((N_DEV - 1,)),
        ],
        compiler_params=pltpu.CompilerParams(collective_id=0),
    )(x_b, Wq_h, Wo_h, K_my, V_my)
